# Initial kernel scaffold; baseline (speedup 1.0000x reference)
import jax
import jax.numpy as jnp
from jax.experimental import pallas as pl


def kernel(x, edge_index, edge_attr, x_emb1, x_emb2, edge_emb1, edge_emb2, W1, b1, W2, b2, gamma, beta):
    raise NotImplementedError("write your pallas kernel here")


# speedup vs baseline: 5.1700x; 5.1700x over previous
import functools

import jax
import jax.numpy as jnp
from jax import lax
from jax.experimental import pallas as pl
from jax.experimental.pallas import tpu as pltpu
from jax.experimental.pallas import tpu_sc as plsc

N = 10000
E = 320000
D = 128
L = 3
NUM_ATOM_TYPE = 120
NUM_CHIRALITY = 3

NC = 2
NS = 16
NW = NC * NS
CHUNK = 128
CPW = 79
EPAD = NW * CPW * CHUNK
RPT = 640
NROWS = NS * RPT
ZR = 16

def _zero_vmem_f32(buf, rows, cols):
    zeros16 = jnp.zeros((16,), jnp.float32)
    for i in range(rows):
        for j in range(cols // 16):
            buf[i, pl.ds(j * 16, 16)] = zeros16


def _scatter_body(width, gather_rows, cpw, h_hbm, src_hbm, dst_hbm, out_hbm,
                  sidx, didx, rows, zbuf, acc, sem):
    cid = lax.axis_index("c")
    sid = lax.axis_index("s")
    wid = sid * NC + cid

    _zero_vmem_f32(zbuf, ZR, width)

    def zloop(k, _):
        pltpu.sync_copy(zbuf, acc.at[pl.ds(sid * RPT + k * ZR, ZR)])
        return 0
    lax.fori_loop(0, RPT // ZR, zloop, 0)
    plsc.subcore_barrier()

    def body(j, _):
        base = pl.multiple_of((wid * cpw + j) * CHUNK, CHUNK)
        pltpu.sync_copy(dst_hbm.at[pl.ds(base, CHUNK)], didx)
        if gather_rows:
            pltpu.sync_copy(src_hbm.at[pl.ds(base, CHUNK)], sidx)
            pltpu.async_copy(h_hbm.at[sidx], rows, sem).wait()
        else:
            pltpu.sync_copy(h_hbm.at[pl.ds(base, CHUNK)], rows)
        pltpu.sync_copy(rows, acc.at[didx], add=True)
        return 0
    lax.fori_loop(0, cpw, body, 0)
    plsc.subcore_barrier()

    def floop(k, _):
        r0 = pl.multiple_of(sid * RPT + k * CHUNK, CHUNK)
        pltpu.sync_copy(acc.at[pl.ds(r0, CHUNK)], rows)
        pltpu.sync_copy(rows, out_hbm.at[pl.ds(cid * NROWS + r0, CHUNK)])
        return 0
    lax.fori_loop(0, RPT // CHUNK, floop, 0)


@functools.lru_cache(maxsize=None)
def _make_scatter(width, gather_rows, cpw=CPW):
    mesh = plsc.VectorSubcoreMesh(
        core_axis_name="c", subcore_axis_name="s", num_cores=NC)
    return functools.partial(
        pl.kernel,
        mesh=mesh,
        out_type=jax.ShapeDtypeStruct((NC * NROWS, width), jnp.float32),
        scratch_types=[
            pltpu.VMEM((CHUNK,), jnp.int32),
            pltpu.VMEM((CHUNK,), jnp.int32),
            pltpu.VMEM((CHUNK, width), jnp.float32),
            pltpu.VMEM((ZR, width), jnp.float32),
            pltpu.VMEM_SHARED((NROWS, width), jnp.float32),
            pltpu.SemaphoreType.DMA,
        ],
    )(functools.partial(_scatter_body, width, gather_rows, cpw))


def _embed_body(x_ref, e1_ref, e2_ref, out_ref):
    i1 = lax.broadcasted_iota(jnp.int32, (N, NUM_ATOM_TYPE), 1)
    oh1 = (i1 == x_ref[:, 0:1]).astype(jnp.float32)
    i2 = lax.broadcasted_iota(jnp.int32, (N, NUM_CHIRALITY), 1)
    oh2 = (i2 == x_ref[:, 1:2]).astype(jnp.float32)
    out_ref[...] = (
        jnp.dot(oh1, e1_ref[...], precision=lax.Precision.HIGHEST)
        + jnp.dot(oh2, e2_ref[...], precision=lax.Precision.HIGHEST)
    )


def _layer_body(last, s_ref, h_ref, cnt_ref, t_ref, const_ref,
                w1_ref, b1_ref, w2_ref, b2_ref, g_ref, beta_ref, out_ref):
    hi = lax.Precision.HIGHEST
    s = s_ref[:N, :] + s_ref[NROWS:NROWS + N, :]
    cnt = cnt_ref[:N, :] + cnt_ref[NROWS:NROWS + N, :]
    aggr = (s + h_ref[...] + const_ref[...]
            + jnp.dot(cnt, t_ref[...], precision=hi))
    hmid = jnp.maximum(jnp.dot(aggr, w1_ref[...]) + b1_ref[...], 0.0)
    hh = jnp.dot(hmid, w2_ref[...]) + b2_ref[...]
    mean = jnp.mean(hh, axis=0, keepdims=True)
    var = jnp.mean((hh - mean) ** 2, axis=0, keepdims=True)
    out = (hh - mean) * lax.rsqrt(var + 1e-5) * g_ref[...] + beta_ref[...]
    if not last:
        out = jnp.maximum(out, 0.0)
    out_ref[...] = out


def kernel(x, edge_index, edge_attr, x_emb1, x_emb2, edge_emb1, edge_emb2,
           W1, b1, W2, b2, gamma, beta):
    src = edge_index[0]
    dst = edge_index[1]
    npad = EPAD - E
    src_p = jnp.concatenate([src, jnp.zeros((npad,), jnp.int32)])
    junk = N + (jnp.arange(npad, dtype=jnp.int32) % (NROWS - N))
    dst_p = jnp.concatenate([dst, junk])

    cols = jnp.arange(D, dtype=jnp.int32)[None, :]
    onehot = ((cols == edge_attr[:, 0:1]).astype(jnp.float32)
              + (cols - 6 == edge_attr[:, 1:2]).astype(jnp.float32))
    onehot_p = jnp.concatenate(
        [onehot, jnp.zeros((npad, D), jnp.float32)], axis=0)

    h = pl.pallas_call(
        _embed_body,
        out_shape=jax.ShapeDtypeStruct((N, D), jnp.float32),
    )(x, x_emb1, x_emb2)

    counts2 = _make_scatter(D, False)(onehot_p, src_p, dst_p)

    for l in range(L):
        t_l = jnp.concatenate(
            [edge_emb1[l], edge_emb2[l], jnp.zeros((D - 9, D), jnp.float32)],
            axis=0)
        const_l = (edge_emb1[l, 4] + edge_emb2[l, 0])[None, :]
        s2 = _make_scatter(D, True)(h, src_p, dst_p)
        h = pl.pallas_call(
            functools.partial(_layer_body, l == L - 1),
            out_shape=jax.ShapeDtypeStruct((N, D), jnp.float32),
        )(s2, h, counts2, t_l, const_l,
          W1[l], b1[l][None, :], W2[l], b2[l][None, :],
          gamma[l][None, :], beta[l][None, :])
    return h
